# baseline (speedup 1.0000x reference)
import functools

import jax
import jax.numpy as jnp
from jax import lax
from jax.experimental import pallas as pl
from jax.experimental.pallas import tpu as pltpu
from jax.experimental.pallas import tpu_sc as plsc

_L = 16
_D = 128
_K = 22


def _hero_body(p_hbm, a_hbm, r_hbm, xb_hbm, prim_hbm, atk_hbm, roles_hbm,
               wt_hbm, b_hbm, out_hbm,
               pidx_v, aidx_v, ridx_v, xb_v, wt_v, b_v,
               prow_v, arow_v, rrows_v, obuf_v,
               sem_idx, sem_par, sem_gat, sem_out):
    is_w0 = jnp.logical_and(lax.axis_index("c") == 0, lax.axis_index("s") == 0)

    @pl.when(is_w0)
    def _():
        cp_p = pltpu.async_copy(p_hbm, pidx_v, sem_idx)
        cp_a = pltpu.async_copy(a_hbm, aidx_v, sem_idx)
        cp_r = pltpu.async_copy(r_hbm, ridx_v, sem_idx)
        cp_x = pltpu.async_copy(xb_hbm, xb_v, sem_par)
        cp_w = pltpu.async_copy(wt_hbm, wt_v, sem_par)
        cp_b = pltpu.async_copy(b_hbm, b_v, sem_par)
        cp_p.wait()
        cp_a.wait()
        cp_r.wait()
        g_p = pltpu.async_copy(prim_hbm.at[pidx_v], prow_v, sem_gat)
        g_a = pltpu.async_copy(atk_hbm.at[aidx_v], arow_v, sem_gat)
        g_r = pltpu.async_copy(roles_hbm.at[ridx_v], rrows_v, sem_gat)

        cp_x.wait()
        cp_w.wait()
        cp_b.wait()
        y = b_v[...]
        for k in range(_K):
            s = pl.ds(k * _L, _L)
            y = y + wt_v[s] * xb_v[s]
        obuf_v[pl.ds(3 * _D, _L)] = y

        g_p.wait()
        g_a.wait()
        g_r.wait()
        for c in range(_D // _L):
            s = pl.ds(c * _L, _L)
            obuf_v[pl.ds(0 * _D + c * _L, _L)] = prow_v[0, s]
            obuf_v[pl.ds(1 * _D + c * _L, _L)] = arow_v[0, s]
            acc = (rrows_v[0, s] + rrows_v[1, s] + rrows_v[2, s]
                   + rrows_v[3, s] + rrows_v[4, s])
            obuf_v[pl.ds(2 * _D + c * _L, _L)] = acc / 5.0
        pltpu.async_copy(obuf_v, out_hbm, sem_out).wait()


_hero_sc = functools.partial(
    pl.kernel,
    out_type=jax.ShapeDtypeStruct((3 * _D + _L,), jnp.float32),
    mesh=plsc.VectorSubcoreMesh(core_axis_name="c", subcore_axis_name="s"),
    scratch_types=[
        pltpu.VMEM((1,), jnp.int32),
        pltpu.VMEM((1,), jnp.int32),
        pltpu.VMEM((5,), jnp.int32),
        pltpu.VMEM((_K * _L,), jnp.float32),
        pltpu.VMEM((_K * _L,), jnp.float32),
        pltpu.VMEM((_L,), jnp.float32),
        pltpu.VMEM((1, _D), jnp.float32),
        pltpu.VMEM((1, _D), jnp.float32),
        pltpu.VMEM((5, _D), jnp.float32),
        pltpu.VMEM((3 * _D + _L,), jnp.float32),
        pltpu.SemaphoreType.DMA,
        pltpu.SemaphoreType.DMA,
        pltpu.SemaphoreType.DMA,
        pltpu.SemaphoreType.DMA,
    ],
)(_hero_body)


def kernel(p_attrs, a_types, role_i, float_stats, primary_table,
           attack_table, roles_table, proj_W, proj_b):
    wt_flat = proj_W.T.reshape(-1)
    xb_flat = jnp.repeat(float_stats, _L)
    return _hero_sc(
        p_attrs.astype(jnp.int32),
        a_types.astype(jnp.int32),
        role_i.astype(jnp.int32),
        xb_flat,
        primary_table,
        attack_table,
        roles_table,
        wt_flat,
        proj_b,
    )


# speedup vs baseline: 1.0931x; 1.0931x over previous
import functools

import jax
import jax.numpy as jnp
from jax import lax
from jax.experimental import pallas as pl
from jax.experimental.pallas import tpu as pltpu
from jax.experimental.pallas import tpu_sc as plsc

_L = 16
_D = 128
_K = 22


def _hero_body(p_hbm, a_hbm, r_hbm, xb_hbm, prim_hbm, atk_hbm, roles_hbm,
               wt_hbm, b_hbm, out_hbm,
               pidx_v, aidx_v, ridx_v, xb_v, wt_v, b_v,
               prow_v, arow_v, rrows_v, obuf_v,
               sem_idx, sem_par, sem_gat, sem_out):
    is_w0 = jnp.logical_and(lax.axis_index("c") == 0, lax.axis_index("s") == 0)

    @pl.when(is_w0)
    def _():
        cp_p = pltpu.async_copy(p_hbm, pidx_v, sem_idx)
        cp_a = pltpu.async_copy(a_hbm, aidx_v, sem_idx)
        cp_r = pltpu.async_copy(r_hbm, ridx_v, sem_idx)
        cp_x = pltpu.async_copy(xb_hbm, xb_v, sem_par)
        cp_w = pltpu.async_copy(wt_hbm, wt_v, sem_par)
        cp_b = pltpu.async_copy(b_hbm, b_v, sem_par)
        cp_p.wait()
        cp_a.wait()
        cp_r.wait()
        g_p = pltpu.async_copy(prim_hbm.at[pidx_v], prow_v, sem_gat)
        g_a = pltpu.async_copy(atk_hbm.at[aidx_v], arow_v, sem_gat)
        g_r = pltpu.async_copy(roles_hbm.at[ridx_v], rrows_v, sem_gat)

        cp_x.wait()
        cp_w.wait()
        cp_b.wait()
        y = b_v[...]
        for k in range(_K):
            s = pl.ds(k * _L, _L)
            y = y + wt_v[s] * xb_v[s]
        obuf_v[pl.ds(3 * _D, _L)] = y

        g_p.wait()
        g_a.wait()
        g_r.wait()
        for c in range(_D // _L):
            s = pl.ds(c * _L, _L)
            obuf_v[pl.ds(0 * _D + c * _L, _L)] = prow_v[0, s]
            obuf_v[pl.ds(1 * _D + c * _L, _L)] = arow_v[0, s]
            acc = (rrows_v[0, s] + rrows_v[1, s] + rrows_v[2, s]
                   + rrows_v[3, s] + rrows_v[4, s])
            obuf_v[pl.ds(2 * _D + c * _L, _L)] = acc / 5.0
        pltpu.async_copy(obuf_v, out_hbm, sem_out).wait()


_hero_sc = functools.partial(
    pl.kernel,
    out_type=jax.ShapeDtypeStruct((3 * _D + _L,), jnp.float32),
    mesh=plsc.VectorSubcoreMesh(core_axis_name="c", subcore_axis_name="s",
                                num_cores=1, num_subcores=1),
    scratch_types=[
        pltpu.VMEM((1,), jnp.int32),
        pltpu.VMEM((1,), jnp.int32),
        pltpu.VMEM((5,), jnp.int32),
        pltpu.VMEM((_K * _L,), jnp.float32),
        pltpu.VMEM((_K * _L,), jnp.float32),
        pltpu.VMEM((_L,), jnp.float32),
        pltpu.VMEM((1, _D), jnp.float32),
        pltpu.VMEM((1, _D), jnp.float32),
        pltpu.VMEM((5, _D), jnp.float32),
        pltpu.VMEM((3 * _D + _L,), jnp.float32),
        pltpu.SemaphoreType.DMA,
        pltpu.SemaphoreType.DMA,
        pltpu.SemaphoreType.DMA,
        pltpu.SemaphoreType.DMA,
    ],
)(_hero_body)


def kernel(p_attrs, a_types, role_i, float_stats, primary_table,
           attack_table, roles_table, proj_W, proj_b):
    wt_flat = proj_W.T.reshape(-1)
    xb_flat = jnp.repeat(float_stats, _L)
    return _hero_sc(
        p_attrs.astype(jnp.int32),
        a_types.astype(jnp.int32),
        role_i.astype(jnp.int32),
        xb_flat,
        primary_table,
        attack_table,
        roles_table,
        wt_flat,
        proj_b,
    )
